# baseline (speedup 1.0000x reference)
import jax
import jax.numpy as jnp
from jax.experimental import pallas as pl


_BLOCK_ROWS = 1024


def _threshold_kernel(x_ref, o_ref):
    o_ref[...] = jnp.where(x_ref[...] > 0.1, 1.0, 0.0).astype(jnp.float32)


def kernel(adj):
    b, c, n, m = adj.shape
    flat = adj.reshape(b * c * n, m)
    rows = flat.shape[0]
    grid = rows // _BLOCK_ROWS
    out = pl.pallas_call(
        _threshold_kernel,
        grid=(grid,),
        in_specs=[pl.BlockSpec((_BLOCK_ROWS, m), lambda i: (i, 0))],
        out_specs=pl.BlockSpec((_BLOCK_ROWS, m), lambda i: (i, 0)),
        out_shape=jax.ShapeDtypeStruct((rows, m), jnp.float32),
    )(flat)
    return out.reshape(b, c, n, m)
